# baseline (speedup 1.0000x reference)
import jax
import jax.numpy as jnp
from jax.experimental import pallas as pl


S_BLK = 512


def _pos_add_kernel(x_ref, pos_ref, out_ref):
    out_ref[...] = x_ref[...] + pos_ref[...][None, :, :]


def _part(x, pos, b0, nb, seq_len, d_model):
    n_blocks = seq_len // S_BLK
    return pl.pallas_call(
        _pos_add_kernel,
        grid=(n_blocks,),
        in_specs=[
            pl.BlockSpec((nb, S_BLK, d_model), lambda s: (b0 // nb, s, 0)),
            pl.BlockSpec((S_BLK, d_model), lambda s: (s, 0)),
        ],
        out_specs=pl.BlockSpec((nb, S_BLK, d_model), lambda s: (0, s, 0)),
        out_shape=jax.ShapeDtypeStruct((nb, seq_len, d_model), x.dtype),
    )(x, pos)


def kernel(x, pos_table):
    batch, seq_len, d_model = x.shape
    pos = pos_table[:seq_len]
    a = _part(x, pos, 0, 2, seq_len, d_model)
    b = _part(x, pos, 2, 2, seq_len, d_model)
    return jnp.concatenate([a, b], axis=0)


# speedup vs baseline: 2.0042x; 2.0042x over previous
import functools

import jax
import jax.numpy as jnp
from jax.experimental import pallas as pl
from jax.experimental.pallas import tpu as pltpu


R = 512
K = 8


def _pos_add_body(x_hbm, pos_hbm, out_hbm, posbuf, xbuf, obuf,
                  pos_sems, rd_sems, wr_sems, *, batch, n_pos_chunks):
    n_steps = n_pos_chunks * batch

    def rows(t):
        b = jax.lax.rem(t, batch)
        c = jax.lax.div(t, batch)
        return b * (n_pos_chunks * R) + c * R

    def pos_copy(c):
        return pltpu.make_async_copy(
            pos_hbm.at[pl.ds(c * R, R)], posbuf.at[jax.lax.rem(c, 2)],
            pos_sems.at[jax.lax.rem(c, 2)])

    def rd_copy(t):
        slot = jax.lax.rem(t, K)
        return pltpu.make_async_copy(
            x_hbm.at[pl.ds(rows(t), R)], xbuf.at[slot], rd_sems.at[slot])

    def wr_copy(t):
        slot = jax.lax.rem(t, K)
        return pltpu.make_async_copy(
            obuf.at[slot], out_hbm.at[pl.ds(rows(t), R)], wr_sems.at[slot])

    pos_copy(0).start()
    pos_copy(1).start()
    for t in range(K):
        rd_copy(t).start()

    def step(t, _):
        slot = jax.lax.rem(t, K)
        b = jax.lax.rem(t, batch)
        c = jax.lax.div(t, batch)
        pslot = jax.lax.rem(c, 2)

        @pl.when(b == 0)
        def _():
            pos_copy(c).wait()

        rd_copy(t).wait()

        @pl.when(t >= K)
        def _():
            wr_copy(t - K).wait()

        obuf[slot] = xbuf[slot] + posbuf[pslot]
        wr_copy(t).start()

        @pl.when(jnp.logical_and(b == batch - 1, c + 2 < n_pos_chunks))
        def _():
            pos_copy(c + 2).start()

        @pl.when(t + K < n_steps)
        def _():
            rd_copy(t + K).start()

        return 0

    jax.lax.fori_loop(0, n_steps, step, 0)

    for j in range(K):
        wr_copy(n_steps - K + j).wait()


def kernel(x, pos_table):
    batch, seq_len, d_model = x.shape
    rows = batch * seq_len
    n_pos_chunks = seq_len // R
    xf = x.reshape(rows, d_model)
    pos = pos_table[:seq_len]

    body = functools.partial(_pos_add_body, batch=batch,
                             n_pos_chunks=n_pos_chunks)
    out = pl.pallas_call(
        body,
        in_specs=[
            pl.BlockSpec(memory_space=pl.ANY),
            pl.BlockSpec(memory_space=pl.ANY),
        ],
        out_specs=pl.BlockSpec(memory_space=pl.ANY),
        out_shape=jax.ShapeDtypeStruct((rows, d_model), x.dtype),
        scratch_shapes=[
            pltpu.VMEM((2, R, d_model), x.dtype),
            pltpu.VMEM((K, R, d_model), x.dtype),
            pltpu.VMEM((K, R, d_model), x.dtype),
            pltpu.SemaphoreType.DMA((2,)),
            pltpu.SemaphoreType.DMA((K,)),
            pltpu.SemaphoreType.DMA((K,)),
        ],
    )(xf, pos)
    return out.reshape(batch, seq_len, d_model)
